# baseline (speedup 1.0000x reference)
import math

import jax
import jax.numpy as jnp
from jax import lax
from jax.experimental import pallas as pl
from jax.experimental.pallas import tpu as pltpu
from jax.experimental.pallas import tpu_sc as plsc

VSZ = 1000000
DSZ = 64
MXLEN = 1000
MAX_TIMESCALE = 10000.0
B = 1024
L = 200

NC = 2
NS = 16
NW = NC * NS
HALF = L // 2
ROWS2 = B * 2
CH = ROWS2 // NW
SCALE = math.sqrt(DSZ)


def _pos_encoding():
    log_inc = math.log(MAX_TIMESCALE) / DSZ
    inv = jnp.exp(jnp.arange(0, DSZ, 2, dtype=jnp.float32) * -log_inc)
    pos = jnp.arange(0, MXLEN, dtype=jnp.float32)[:, None]
    pe = jnp.zeros((MXLEN, DSZ), jnp.float32)
    pe = pe.at[:, 0::2].set(jnp.sin(pos * inv))
    pe = pe.at[:, 1::2].set(jnp.cos(pos * inv))
    return pe[:L]


def _sc_body(x_hbm, pe_hbm, table_hbm, out_hbm, idx_v, pe_v, rows_v, gsem):
    wid = lax.axis_index("s") * NC + lax.axis_index("c")
    base = wid * CH
    pltpu.sync_copy(x_hbm.at[pl.ds(base, CH), :], idx_v)
    pltpu.sync_copy(pe_hbm, pe_v)

    @pl.loop(0, CH, step=2)
    def _chunks(j):
        for b in range(2):
            jj = j + b
            pltpu.async_copy(table_hbm.at[idx_v.at[jj]], rows_v.at[b], gsem).wait()

            @pl.loop(0, HALF)
            def _rows(r):
                for c4 in range(DSZ // 16):
                    sl = pl.ds(c4 * 16, 16)
                    rows_v[b, r, sl] = rows_v[b, r, sl] * SCALE + pe_v[b * HALF + r, sl]

            pltpu.sync_copy(rows_v.at[b], out_hbm.at[base + jj])


def kernel(x, table):
    pe = _pos_encoding()
    x2 = x.reshape(ROWS2, HALF)

    run = pl.kernel(
        _sc_body,
        out_type=jax.ShapeDtypeStruct((ROWS2, HALF, DSZ), jnp.float32),
        mesh=plsc.VectorSubcoreMesh(core_axis_name="c", subcore_axis_name="s"),
        scratch_types=[
            pltpu.VMEM((CH, HALF), jnp.int32),
            pltpu.VMEM((L, DSZ), jnp.float32),
            pltpu.VMEM((2, HALF, DSZ), jnp.float32),
            pltpu.SemaphoreType.DMA,
        ],
        compiler_params=pltpu.CompilerParams(use_tc_tiling_on_sc=False),
    )
    out = run(x2, pe, table)
    return out.reshape(B, L, DSZ)


# speedup vs baseline: 1.0637x; 1.0637x over previous
import math

import jax
import jax.numpy as jnp
from jax import lax
from jax.experimental import pallas as pl
from jax.experimental.pallas import tpu as pltpu
from jax.experimental.pallas import tpu_sc as plsc

VSZ = 1000000
DSZ = 64
MXLEN = 1000
MAX_TIMESCALE = 10000.0
B = 1024
L = 200

NC = 2
NS = 16
NW = NC * NS
HALF = L // 2
ROWS2 = B * 2
CH = ROWS2 // NW
SCALE = math.sqrt(DSZ)

NBUF = 8
AHEAD = 4


def _pos_encoding():
    log_inc = math.log(MAX_TIMESCALE) / DSZ
    inv = jnp.exp(jnp.arange(0, DSZ, 2, dtype=jnp.float32) * -log_inc)
    pos = jnp.arange(0, MXLEN, dtype=jnp.float32)[:, None]
    pe = jnp.zeros((MXLEN, DSZ), jnp.float32)
    pe = pe.at[:, 0::2].set(jnp.sin(pos * inv))
    pe = pe.at[:, 1::2].set(jnp.cos(pos * inv))
    return pe[:L].reshape(2, HALF, DSZ)


def _sc_body(x_hbm, pe_hbm, table_hbm, out_hbm, idx_v, pe_v, rows_v, *sems):
    gs = sems[:NBUF]
    ws = sems[NBUF:]
    wid = lax.axis_index("s") * NC + lax.axis_index("c")
    base = wid * CH
    pltpu.sync_copy(x_hbm.at[pl.ds(base, CH), :], idx_v)
    pltpu.sync_copy(pe_hbm, pe_v)

    def fire_gather(jj, b):
        pltpu.async_copy(table_hbm.at[idx_v.at[jj]], rows_v.at[b], gs[b])

    def wait_gather(jj, b):
        pltpu.make_async_copy(table_hbm.at[idx_v.at[jj]], rows_v.at[b], gs[b]).wait()

    def fire_write(jj, b):
        pltpu.async_copy(rows_v.at[b], out_hbm.at[base + jj], ws[b])

    def wait_write(b):
        pltpu.make_async_copy(rows_v.at[b], out_hbm.at[base], ws[b]).wait()

    for k in range(AHEAD):
        fire_gather(k, k)

    @pl.loop(0, CH, step=NBUF)
    def _grp(j):
        for b in range(NBUF):
            jj = j + b
            wait_gather(jj, b)
            par = b % 2

            @pl.loop(0, HALF, unroll=4)
            def _rows(r):
                for c4 in range(DSZ // 16):
                    sl = pl.ds(c4 * 16, 16)
                    rows_v[b, r, sl] = rows_v[b, r, sl] * SCALE + pe_v[par, r, sl]

            b2 = (b + AHEAD) % NBUF

            @pl.when(jj >= NBUF - AHEAD)
            def _():
                wait_write(b2)

            @pl.when(jj + AHEAD < CH)
            def _():
                fire_gather(jj + AHEAD, b2)

            fire_write(jj, b)

    for k in range(1, AHEAD + 1):
        wait_write((CH - k) % NBUF)


def kernel(x, table):
    pe = _pos_encoding()
    x2 = x.reshape(ROWS2, HALF)

    run = pl.kernel(
        _sc_body,
        out_type=jax.ShapeDtypeStruct((ROWS2, HALF, DSZ), jnp.float32),
        mesh=plsc.VectorSubcoreMesh(core_axis_name="c", subcore_axis_name="s"),
        scratch_types=[
            pltpu.VMEM((CH, HALF), jnp.int32),
            pltpu.VMEM((2, HALF, DSZ), jnp.float32),
            pltpu.VMEM((NBUF, HALF, DSZ), jnp.float32),
        ]
        + [pltpu.SemaphoreType.DMA] * (2 * NBUF),
        compiler_params=pltpu.CompilerParams(use_tc_tiling_on_sc=False),
    )
    out = run(x2, pe, table)
    return out.reshape(B, L, DSZ)
